# Initial kernel scaffold; baseline (speedup 1.0000x reference)
import jax
import jax.numpy as jnp
from jax.experimental import pallas as pl


def kernel(seq_length, relative_positional_encoding):
    raise NotImplementedError("write your pallas kernel here")


# speedup vs baseline: 4.5187x; 4.5187x over previous
import jax
import jax.numpy as jnp
from jax.experimental import pallas as pl

NUM_HEADS = 12
SEQ = 256
HEAD_DIM = 64
ROWS_PER_STEP = 8


def _copy_kernel(rev_ref, out_ref):
    i0 = pl.program_id(0) * ROWS_PER_STEP
    for di in range(ROWS_PER_STEP):
        out_ref[:, di, :, :] = rev_ref[:, pl.ds(SEQ - (i0 + di), SEQ), :]


def kernel(seq_length, relative_positional_encoding):
    start = seq_length - SEQ
    sl = jax.lax.dynamic_slice(
        relative_positional_encoding,
        (0, start, 0),
        (NUM_HEADS, 2 * SEQ, HEAD_DIM),
    )
    rev = sl[:, ::-1, :]

    grid = (SEQ // ROWS_PER_STEP,)
    return pl.pallas_call(
        _copy_kernel,
        grid=grid,
        in_specs=[
            pl.BlockSpec((NUM_HEADS, 2 * SEQ, HEAD_DIM), lambda i: (0, 0, 0)),
        ],
        out_specs=pl.BlockSpec(
            (NUM_HEADS, ROWS_PER_STEP, SEQ, HEAD_DIM), lambda i: (0, i, 0, 0)
        ),
        out_shape=jax.ShapeDtypeStruct(
            (NUM_HEADS, SEQ, SEQ, HEAD_DIM), jnp.float32
        ),
    )(rev)
